# baseline (device time: 507681 ns/iter reference)
import jax
import jax.numpy as jnp
from jax import lax
from jax.experimental import pallas as pl
from jax.experimental.pallas import tpu as pltpu

K = 16
S = 4
PL = 16
SL = 4


def kernel(x):
    m, n = x.shape
    n_out = n // 2
    h = m // 2
    c = h // K
    rb = m // PL
    m_total = 2 * m

    def body(x_ref, out_ref, stage_ref, loc_ref, stage_sems, rd_sems,
             wr_sems, send1_sems, recv1_sems, send2_sems, recv2_sems):
        my_x = lax.axis_index("x")
        my_y = lax.axis_index("y")
        peer_y = 1 - my_y
        y_peer = (my_x, peer_y)
        x_nbr = (1 - my_x, my_y)

        barrier_sem = pltpu.get_barrier_semaphore()
        for nbr in (y_peer, x_nbr):
            pl.semaphore_signal(
                barrier_sem, inc=1,
                device_id=nbr, device_id_type=pl.DeviceIdType.MESH,
            )
        pl.semaphore_wait(barrier_sem, 2)

        def rd_loc(k):
            return pltpu.make_async_copy(
                x_ref.at[pl.ds(k * rb, rb), pl.ds(my_y * n_out, n_out)],
                loc_ref.at[k % SL],
                rd_sems.at[k % SL],
            )

        def wr_loc(k):
            return pltpu.make_async_copy(
                loc_ref.at[k % SL],
                out_ref.at[pl.ds(my_y * m + k * rb, rb), :],
                wr_sems.at[k % SL],
            )

        def stage(k):
            return pltpu.make_async_copy(
                x_ref.at[pl.ds(my_x * h + k * c, c),
                         pl.ds(peer_y * n_out, n_out)],
                stage_ref.at[k % S],
                stage_sems.at[k % S],
            )

        def p1_rdma(k):
            return pltpu.make_async_remote_copy(
                src_ref=stage_ref.at[k % S],
                dst_ref=out_ref.at[pl.ds(my_y * m + my_x * h + k * c, c), :],
                send_sem=send1_sems.at[k],
                recv_sem=recv1_sems.at[k],
                device_id=y_peer,
                device_id_type=pl.DeviceIdType.MESH,
            )

        def p1_recv(k):
            return pltpu.make_async_remote_copy(
                src_ref=stage_ref.at[k % S],
                dst_ref=out_ref.at[pl.ds(peer_y * m + my_x * h + k * c, c), :],
                send_sem=send1_sems.at[k],
                recv_sem=recv1_sems.at[k],
                device_id=y_peer,
                device_id_type=pl.DeviceIdType.MESH,
            )

        def p2_rdma(k):
            rows = pl.ds(peer_y * m + my_x * h + k * c, c)
            return pltpu.make_async_remote_copy(
                src_ref=out_ref.at[rows, :],
                dst_ref=out_ref.at[rows, :],
                send_sem=send2_sems.at[k],
                recv_sem=recv2_sems.at[k],
                device_id=x_nbr,
                device_id_type=pl.DeviceIdType.MESH,
            )

        def p2_recv(k):
            rows = pl.ds(peer_y * m + (1 - my_x) * h + k * c, c)
            return pltpu.make_async_remote_copy(
                src_ref=out_ref.at[rows, :],
                dst_ref=out_ref.at[rows, :],
                send_sem=send2_sems.at[k],
                recv_sem=recv2_sems.at[k],
                device_id=x_nbr,
                device_id_type=pl.DeviceIdType.MESH,
            )

        stage(0).start()
        for k in range(SL):
            rd_loc(k).start()

        for k in range(K):
            stage(k).wait()
            p1_rdma(k).start()
            j = k + 1
            if j < K:
                if j >= S:
                    p1_rdma(j - S).wait_send()
                stage(j).start()
            rd_loc(k).wait()
            wr_loc(k).start()
            jl = k + SL
            if jl < PL:
                wr_loc(jl - SL).wait()
                rd_loc(jl).start()
            p1_recv(k).wait_recv()
            p2_rdma(k).start()

        for k in range(K - S, K):
            p1_rdma(k).wait_send()
        for k in range(PL - SL, PL):
            wr_loc(k).wait()
        for k in range(K):
            p2_recv(k).wait_recv()
            p2_rdma(k).wait_send()

    return pl.pallas_call(
        body,
        out_shape=jax.ShapeDtypeStruct((m_total, n_out), x.dtype),
        in_specs=[pl.BlockSpec(memory_space=pl.ANY)],
        out_specs=pl.BlockSpec(memory_space=pl.ANY),
        scratch_shapes=[
            pltpu.VMEM((S, c, n_out), x.dtype),
            pltpu.VMEM((SL, rb, n_out), x.dtype),
            pltpu.SemaphoreType.DMA((S,)),
            pltpu.SemaphoreType.DMA((SL,)),
            pltpu.SemaphoreType.DMA((SL,)),
            pltpu.SemaphoreType.DMA((K,)),
            pltpu.SemaphoreType.DMA((K,)),
            pltpu.SemaphoreType.DMA((K,)),
            pltpu.SemaphoreType.DMA((K,)),
        ],
        compiler_params=pltpu.CompilerParams(collective_id=0),
    )(x)


# device time: 477402 ns/iter; 1.0634x vs baseline; 1.0634x over previous
import jax
import jax.numpy as jnp
from jax import lax
from jax.experimental import pallas as pl
from jax.experimental.pallas import tpu as pltpu

CHUNK_ROWS = [512] * 15 + [256] * 2
PL = 16
SL = 4


def kernel(x):
    m, n = x.shape
    n_out = n // 2
    h = m // 2
    rb = m // PL
    m_total = 2 * m

    assert sum(CHUNK_ROWS) == h
    K = len(CHUNK_ROWS)
    offs = [sum(CHUNK_ROWS[:k]) for k in range(K)]

    def body(x_ref, out_ref, stage_ref, loc_ref, stage_sems, rd_sems,
             wr_sems, send1_sems, recv1_sems, send2_sems, recv2_sems):
        my_x = lax.axis_index("x")
        my_y = lax.axis_index("y")
        peer_y = 1 - my_y
        y_peer = (my_x, peer_y)
        x_nbr = (1 - my_x, my_y)

        def rd_loc(k):
            return pltpu.make_async_copy(
                x_ref.at[pl.ds(k * rb, rb), pl.ds(my_y * n_out, n_out)],
                loc_ref.at[k % SL],
                rd_sems.at[k % SL],
            )

        def wr_loc(k):
            return pltpu.make_async_copy(
                loc_ref.at[k % SL],
                out_ref.at[pl.ds(my_y * m + k * rb, rb), :],
                wr_sems.at[k % SL],
            )

        def stage(k):
            sz = CHUNK_ROWS[k]
            return pltpu.make_async_copy(
                x_ref.at[pl.ds(my_x * h + offs[k], sz),
                         pl.ds(peer_y * n_out, n_out)],
                stage_ref.at[pl.ds(offs[k], sz), :],
                stage_sems.at[k],
            )

        def p1_rdma(k):
            sz = CHUNK_ROWS[k]
            return pltpu.make_async_remote_copy(
                src_ref=stage_ref.at[pl.ds(offs[k], sz), :],
                dst_ref=out_ref.at[pl.ds(my_y * m + my_x * h + offs[k], sz), :],
                send_sem=send1_sems.at[k],
                recv_sem=recv1_sems.at[k],
                device_id=y_peer,
                device_id_type=pl.DeviceIdType.MESH,
            )

        def p1_recv(k):
            sz = CHUNK_ROWS[k]
            return pltpu.make_async_remote_copy(
                src_ref=stage_ref.at[pl.ds(offs[k], sz), :],
                dst_ref=out_ref.at[pl.ds(peer_y * m + my_x * h + offs[k], sz), :],
                send_sem=send1_sems.at[k],
                recv_sem=recv1_sems.at[k],
                device_id=y_peer,
                device_id_type=pl.DeviceIdType.MESH,
            )

        def p2_rdma(k):
            sz = CHUNK_ROWS[k]
            rows = pl.ds(peer_y * m + my_x * h + offs[k], sz)
            return pltpu.make_async_remote_copy(
                src_ref=out_ref.at[rows, :],
                dst_ref=out_ref.at[rows, :],
                send_sem=send2_sems.at[k],
                recv_sem=recv2_sems.at[k],
                device_id=x_nbr,
                device_id_type=pl.DeviceIdType.MESH,
            )

        def p2_recv(k):
            sz = CHUNK_ROWS[k]
            rows = pl.ds(peer_y * m + (1 - my_x) * h + offs[k], sz)
            return pltpu.make_async_remote_copy(
                src_ref=out_ref.at[rows, :],
                dst_ref=out_ref.at[rows, :],
                send_sem=send2_sems.at[k],
                recv_sem=recv2_sems.at[k],
                device_id=x_nbr,
                device_id_type=pl.DeviceIdType.MESH,
            )

        for k in range(K):
            stage(k).start()
        for k in range(SL):
            rd_loc(k).start()

        barrier_sem = pltpu.get_barrier_semaphore()
        for nbr in (y_peer, x_nbr):
            pl.semaphore_signal(
                barrier_sem, inc=1,
                device_id=nbr, device_id_type=pl.DeviceIdType.MESH,
            )
        pl.semaphore_wait(barrier_sem, 2)

        for k in range(K):
            stage(k).wait()
            p1_rdma(k).start()

        for k in range(K):
            p1_recv(k).wait_recv()
            p2_rdma(k).start()
            for kl in range(k * PL // K, (k + 1) * PL // K):
                rd_loc(kl).wait()
                wr_loc(kl).start()
                jl = kl + SL
                if jl < PL:
                    wr_loc(jl - SL).wait()
                    rd_loc(jl).start()

        for k in range(K):
            p1_rdma(k).wait_send()
        for k in range(PL - SL, PL):
            wr_loc(k).wait()
        for k in range(K):
            p2_recv(k).wait_recv()
            p2_rdma(k).wait_send()

    return pl.pallas_call(
        body,
        out_shape=jax.ShapeDtypeStruct((m_total, n_out), x.dtype),
        in_specs=[pl.BlockSpec(memory_space=pl.ANY)],
        out_specs=pl.BlockSpec(memory_space=pl.ANY),
        scratch_shapes=[
            pltpu.VMEM((h, n_out), x.dtype),
            pltpu.VMEM((SL, rb, n_out), x.dtype),
            pltpu.SemaphoreType.DMA((K,)),
            pltpu.SemaphoreType.DMA((SL,)),
            pltpu.SemaphoreType.DMA((SL,)),
            pltpu.SemaphoreType.DMA((K,)),
            pltpu.SemaphoreType.DMA((K,)),
            pltpu.SemaphoreType.DMA((K,)),
            pltpu.SemaphoreType.DMA((K,)),
        ],
        compiler_params=pltpu.CompilerParams(
            collective_id=0, vmem_limit_bytes=60 * 1024 * 1024,
        ),
    )(x)
